# baseline (speedup 1.0000x reference)
import functools

import jax
import jax.numpy as jnp
from jax import lax
from jax.experimental import pallas as pl
from jax.experimental.pallas import tpu as pltpu
from jax.experimental.pallas import tpu_sc as plsc

N = 1048576
X_LIM, Y_LIM, Z_LIM = 511, 511, 255

NC, NS = 2, 16
NW = NC * NS
PW = N // NW
S = 8192
NSUB = PW // S
VPC = S // 16

_mesh = plsc.VectorSubcoreMesh(core_axis_name="c", subcore_axis_name="s")


@functools.partial(
    pl.kernel,
    mesh=_mesh,
    compiler_params=pltpu.CompilerParams(needs_layout_passes=False),
    out_type=jax.ShapeDtypeStruct((4 * N,), jnp.float32),
    scratch_types=[
        pltpu.VMEM((3 * S,), jnp.float32),
        pltpu.VMEM((S,), jnp.int32),
        pltpu.VMEM((S,), jnp.float32),
        pltpu.VMEM((S,), jnp.float32),
        pltpu.VMEM((4 * S,), jnp.float32),
        pltpu.SemaphoreType.DMA,
    ],
)
def _ct_gather(xyz_hbm, img_hbm, out_hbm, xyz_v, idx_v, mul_v, sig_v,
               out_v, sem):
    wid = lax.axis_index("s") * NC + lax.axis_index("c")
    iota = lax.iota(jnp.int32, 16)
    ones16 = jnp.full((16,), 1.0, jnp.float32)

    def _fill(g, c):
        out_v[pl.ds(g * 16, 16)] = ones16
        return c
    lax.fori_loop(0, (4 * S) // 16, _fill, 0)

    def _sub(sub, c):
        sbase = wid * PW + sub * S
        pltpu.sync_copy(xyz_hbm.at[pl.ds(3 * sbase, 3 * S)], xyz_v)

        def _comp(g, cc):
            xi = (g * 16 + iota) * 3
            x = plsc.load_gather(xyz_v, [xi])
            y = plsc.load_gather(xyz_v, [xi + 1])
            z = plsc.load_gather(xyz_v, [xi + 2])
            ix = (((x + 1.0) * 0.5) * 511.0).astype(jnp.int32)
            iy = (((y + 1.0) * 0.5) * 511.0).astype(jnp.int32)
            iz = (((z + 1.0) * 0.5) * 255.0).astype(jnp.int32)
            bad = ((ix < 0) | (iy < 0) | (iz < 0)
                   | (ix > X_LIM) | (iy > Y_LIM) | (iz > Z_LIM))
            lin = ix * ((Y_LIM + 1) * (Z_LIM + 1)) + iy * (Z_LIM + 1) + iz
            idx_v[pl.ds(g * 16, 16)] = jnp.where(bad, 0, lin)
            mul_v[pl.ds(g * 16, 16)] = jnp.where(bad, 0.0, 1.0)
            return cc
        lax.fori_loop(0, VPC, _comp, 0)

        pltpu.async_copy(img_hbm.at[idx_v], sig_v, sem).wait()

        def _outp(g, cc):
            li = g * 16 + iota
            sv = sig_v[pl.ds(g * 16, 16)] * mul_v[pl.ds(g * 16, 16)]
            plsc.store_scatter(out_v, [li * 4 + 3], sv)
            return cc
        lax.fori_loop(0, VPC, _outp, 0)

        pltpu.sync_copy(out_v, out_hbm.at[pl.ds(4 * sbase, 4 * S)])
        return c
    lax.fori_loop(0, NSUB, _sub, 0)


def kernel(xyz, img):
    out = _ct_gather(xyz.reshape(-1), img.reshape(-1))
    return out.reshape(1, N, 4)


# speedup vs baseline: 1.6466x; 1.6466x over previous
import functools

import jax
import jax.numpy as jnp
from jax import lax
from jax.experimental import pallas as pl
from jax.experimental.pallas import tpu as pltpu
from jax.experimental.pallas import tpu_sc as plsc

N = 1048576
X_LIM, Y_LIM, Z_LIM = 511, 511, 255

NC, NS = 2, 16
NW = NC * NS
PW = N // NW
S = 8192
NSUB = PW // S
VPC = S // 16

_mesh = plsc.VectorSubcoreMesh(core_axis_name="c", subcore_axis_name="s")


@functools.partial(
    pl.kernel,
    mesh=_mesh,
    compiler_params=pltpu.CompilerParams(needs_layout_passes=False),
    out_type=jax.ShapeDtypeStruct((4 * N,), jnp.float32),
    scratch_types=[
        pltpu.VMEM((S,), jnp.float32),
        pltpu.VMEM((S,), jnp.float32),
        pltpu.VMEM((S,), jnp.float32),
        pltpu.VMEM((S,), jnp.int32),
        pltpu.VMEM((S,), jnp.float32),
        pltpu.VMEM((S,), jnp.float32),
        pltpu.VMEM((4 * S,), jnp.float32),
        pltpu.SemaphoreType.DMA,
    ],
)
def _ct_gather(xyz_hbm, img_hbm, out_hbm, x_v, y_v, z_v, idx_v, mul_v,
               sig_v, out_v, sem):
    wid = lax.axis_index("s") * NC + lax.axis_index("c")
    ones16 = jnp.full((16,), 1.0, jnp.float32)

    def _fill(g, c):
        out_v[pl.ds(g * 16, 16)] = ones16
        return c
    lax.fori_loop(0, (4 * S) // 16, _fill, 0)

    def _sub(sub, c):
        sbase = wid * PW + sub * S
        pltpu.sync_copy(xyz_hbm.at[pl.ds(sbase, S)], x_v)
        pltpu.sync_copy(xyz_hbm.at[pl.ds(N + sbase, S)], y_v)
        pltpu.sync_copy(xyz_hbm.at[pl.ds(2 * N + sbase, S)], z_v)

        def _comp(g, cc):
            x = x_v[pl.ds(g * 16, 16)]
            y = y_v[pl.ds(g * 16, 16)]
            z = z_v[pl.ds(g * 16, 16)]
            ix = (((x + 1.0) * 0.5) * 511.0).astype(jnp.int32)
            iy = (((y + 1.0) * 0.5) * 511.0).astype(jnp.int32)
            iz = (((z + 1.0) * 0.5) * 255.0).astype(jnp.int32)
            bad = ((ix < 0) | (iy < 0) | (iz < 0)
                   | (ix > X_LIM) | (iy > Y_LIM) | (iz > Z_LIM))
            phys = ((ix << 17) + ((iy >> 3) << 11) + ((iz >> 7) << 10)
                    + ((iy & 7) << 7) + (iz & 127))
            idx_v[pl.ds(g * 16, 16)] = jnp.where(bad, 0, phys)
            mul_v[pl.ds(g * 16, 16)] = jnp.where(bad, 0.0, 1.0)
            return cc
        lax.fori_loop(0, VPC, _comp, 0)

        pltpu.async_copy(img_hbm.at[idx_v], sig_v, sem).wait()

        def _outp(g, cc):
            sv = sig_v[pl.ds(g * 16, 16)] * mul_v[pl.ds(g * 16, 16)]
            b = g * 16
            out_v[pl.ds((b >> 7) * 512 + 384 + (b & 127), 16)] = sv
            return cc
        lax.fori_loop(0, VPC, _outp, 0)

        pltpu.sync_copy(out_v, out_hbm.at[pl.ds(4 * sbase, 4 * S)])
        return c
    lax.fori_loop(0, NSUB, _sub, 0)


def kernel(xyz, img):
    xyz_planar = jnp.transpose(xyz, (2, 0, 1)).reshape(3 * N)
    img_tiled = (img.reshape(512, 64, 8, 2, 128)
                 .transpose(0, 1, 3, 2, 4).reshape(64 * N))
    out = _ct_gather(xyz_planar, img_tiled)
    return out.reshape(N // 128, 4, 128).transpose(0, 2, 1).reshape(1, N, 4)


# speedup vs baseline: 17.0618x; 10.3621x over previous
import functools

import jax
import jax.numpy as jnp
from jax import lax
from jax.experimental import pallas as pl
from jax.experimental.pallas import tpu as pltpu
from jax.experimental.pallas import tpu_sc as plsc

N = 1048576
X_LIM, Y_LIM, Z_LIM = 511, 511, 255

NC, NS = 2, 16
NW = NC * NS
PW = N // NW
S = 8192
NSUB = PW // S
VPC = S // 16
C = 512

_mesh = plsc.VectorSubcoreMesh(core_axis_name="c", subcore_axis_name="s")


@functools.partial(
    pl.kernel,
    mesh=_mesh,
    compiler_params=pltpu.CompilerParams(needs_layout_passes=False),
    out_type=jax.ShapeDtypeStruct((4 * N,), jnp.float32),
    scratch_types=[
        pltpu.VMEM((S,), jnp.float32),
        pltpu.VMEM((S,), jnp.float32),
        pltpu.VMEM((S,), jnp.float32),
        pltpu.VMEM((S + 16,), jnp.int32),
        pltpu.VMEM((S + 16,), jnp.int32),
        pltpu.VMEM((S,), jnp.float32),
        pltpu.VMEM((4 * S,), jnp.float32),
        pltpu.SemaphoreType.DMA,
    ],
)
def _ct_gather(xyz_hbm, img_hbm, out_hbm, x_v, y_v, z_v, cidx_v, cpos_v,
               sig_v, out_v, sem):
    wid = lax.axis_index("s") * NC + lax.axis_index("c")
    iota = lax.iota(jnp.int32, 16)
    ones16 = jnp.full((16,), 1.0, jnp.float32)
    zeros16 = jnp.full((16,), 0.0, jnp.float32)
    zeros16i = jnp.full((16,), 0, jnp.int32)

    def _fill(g, c):
        out_v[pl.ds(g * 16, 16)] = ones16
        return c
    lax.fori_loop(0, (4 * S) // 16, _fill, 0)

    def _fill0(g, c):
        cidx_v[pl.ds(g * 16, 16)] = zeros16i
        return c
    lax.fori_loop(0, (S + 16) // 16, _fill0, 0)

    def _sub(sub, c):
        sbase = wid * PW + sub * S
        pltpu.sync_copy(xyz_hbm.at[pl.ds(sbase, S)], x_v)
        pltpu.sync_copy(xyz_hbm.at[pl.ds(N + sbase, S)], y_v)
        pltpu.sync_copy(xyz_hbm.at[pl.ds(2 * N + sbase, S)], z_v)

        def _comp(g, off):
            x = x_v[pl.ds(g * 16, 16)]
            y = y_v[pl.ds(g * 16, 16)]
            z = z_v[pl.ds(g * 16, 16)]
            ix = (((x + 1.0) * 0.5) * 511.0).astype(jnp.int32)
            iy = (((y + 1.0) * 0.5) * 511.0).astype(jnp.int32)
            iz = (((z + 1.0) * 0.5) * 255.0).astype(jnp.int32)
            bad = ((ix < 0) | (iy < 0) | (iz < 0)
                   | (ix > X_LIM) | (iy > Y_LIM) | (iz > Z_LIM))
            good = jnp.logical_not(bad)
            phys = ((ix << 17) + ((iy >> 3) << 11) + ((iz >> 7) << 10)
                    + ((iy & 7) << 7) + (iz & 127))
            plsc.store_compressed(cidx_v.at[pl.ds(off, 16)], phys,
                                  mask=good)
            plsc.store_compressed(cpos_v.at[pl.ds(off, 16)], g * 16 + iota,
                                  mask=good)
            return off + jnp.max(plsc.all_reduce_population_count(good))
        n_valid = lax.fori_loop(0, VPC, _comp, jnp.int32(0))

        def _zero(g, cc):
            b = g * 16
            out_v[pl.ds((b >> 7) * 512 + 384 + (b & 127), 16)] = zeros16
            return cc
        lax.fori_loop(0, VPC, _zero, 0)

        nch = (n_valid + (C - 1)) // C

        def _gath(j, cc):
            pltpu.async_copy(img_hbm.at[cidx_v.at[pl.ds(j * C, C)]],
                             sig_v.at[pl.ds(j * C, C)], sem).wait()
            return cc
        lax.fori_loop(0, nch, _gath, 0)

        nvec = (n_valid + 15) >> 4

        def _outp(g, cc):
            sv = sig_v[pl.ds(g * 16, 16)]
            pos = cpos_v[pl.ds(g * 16, 16)]
            slot = ((pos >> 7) << 9) + 384 + (pos & 127)
            ok = (g * 16 + iota) < n_valid
            plsc.store_scatter(out_v, [slot], sv, mask=ok)
            return cc
        lax.fori_loop(0, nvec, _outp, 0)

        pltpu.sync_copy(out_v, out_hbm.at[pl.ds(4 * sbase, 4 * S)])
        return c
    lax.fori_loop(0, NSUB, _sub, 0)


def kernel(xyz, img):
    xyz_planar = jnp.transpose(xyz, (2, 0, 1)).reshape(3 * N)
    img_tiled = (img.reshape(512, 64, 8, 2, 128)
                 .transpose(0, 1, 3, 2, 4).reshape(64 * N))
    out = _ct_gather(xyz_planar, img_tiled)
    return out.reshape(N // 128, 4, 128).transpose(0, 2, 1).reshape(1, N, 4)


# speedup vs baseline: 17.2955x; 1.0137x over previous
import functools

import jax
import jax.numpy as jnp
from jax import lax
from jax.experimental import pallas as pl
from jax.experimental.pallas import tpu as pltpu
from jax.experimental.pallas import tpu_sc as plsc

N = 1048576
X_LIM, Y_LIM, Z_LIM = 511, 511, 255

NC, NS = 2, 16
NW = NC * NS
PW = N // NW
S = 8192
NSUB = PW // S
VPC = S // 16
C = 512

_mesh = plsc.VectorSubcoreMesh(core_axis_name="c", subcore_axis_name="s")


@functools.partial(
    pl.kernel,
    mesh=_mesh,
    compiler_params=pltpu.CompilerParams(needs_layout_passes=False),
    out_type=jax.ShapeDtypeStruct((4 * N,), jnp.float32),
    scratch_types=[
        pltpu.VMEM((S,), jnp.float32),
        pltpu.VMEM((S,), jnp.float32),
        pltpu.VMEM((S,), jnp.float32),
        pltpu.VMEM((S + 16,), jnp.int32),
        pltpu.VMEM((S + 16,), jnp.int32),
        pltpu.VMEM((S,), jnp.float32),
        pltpu.VMEM((4 * S,), jnp.float32),
        pltpu.SemaphoreType.DMA,
    ],
)
def _ct_gather(xyz_hbm, img_hbm, out_hbm, x_v, y_v, z_v, cidx_v, cpos_v,
               sig_v, out_v, sem):
    wid = lax.axis_index("s") * NC + lax.axis_index("c")
    iota = lax.iota(jnp.int32, 16)
    ones16 = jnp.full((16,), 1.0, jnp.float32)
    zeros16 = jnp.full((16,), 0.0, jnp.float32)
    zeros16i = jnp.full((16,), 0, jnp.int32)

    def _fill(g, c):
        out_v[pl.ds(g * 16, 16)] = ones16
        return c
    lax.fori_loop(0, (4 * S) // 16, _fill, 0)

    def _fill0(g, c):
        cidx_v[pl.ds(g * 16, 16)] = zeros16i
        return c
    lax.fori_loop(0, (S + 16) // 16, _fill0, 0)

    def _sub(sub, c):
        sbase = wid * PW + sub * S
        pltpu.sync_copy(xyz_hbm.at[pl.ds(sbase, S)], x_v)
        pltpu.sync_copy(xyz_hbm.at[pl.ds(N + sbase, S)], y_v)
        pltpu.sync_copy(xyz_hbm.at[pl.ds(2 * N + sbase, S)], z_v)

        def _comp(g, off):
            x = x_v[pl.ds(g * 16, 16)]
            y = y_v[pl.ds(g * 16, 16)]
            z = z_v[pl.ds(g * 16, 16)]
            ix = (((x + 1.0) * 0.5) * 511.0).astype(jnp.int32)
            iy = (((y + 1.0) * 0.5) * 511.0).astype(jnp.int32)
            iz = (((z + 1.0) * 0.5) * 255.0).astype(jnp.int32)
            bad = ((ix < 0) | (iy < 0) | (iz < 0)
                   | (ix > X_LIM) | (iy > Y_LIM) | (iz > Z_LIM))
            good = jnp.logical_not(bad)
            phys = ((ix << 17) + ((iy >> 3) << 11) + ((iz >> 7) << 10)
                    + ((iy & 7) << 7) + (iz & 127))
            plsc.store_compressed(cidx_v.at[pl.ds(off, 16)], phys,
                                  mask=good)
            plsc.store_compressed(cpos_v.at[pl.ds(off, 16)], g * 16 + iota,
                                  mask=good)
            return off + jnp.max(plsc.all_reduce_population_count(good))
        n_valid = lax.fori_loop(0, VPC, _comp, jnp.int32(0))

        def _zero(g, cc):
            b = g * 16
            out_v[pl.ds((b >> 7) * 512 + 384 + (b & 127), 16)] = zeros16
            return cc
        lax.fori_loop(0, VPC, _zero, 0)

        nch = (n_valid + (C - 1)) // C

        def _fire(j, cc):
            pltpu.async_copy(img_hbm.at[cidx_v.at[pl.ds(j * C, C)]],
                             sig_v.at[pl.ds(j * C, C)], sem)
            return cc
        lax.fori_loop(0, nch, _fire, 0)

        def _drain(j, cc):
            pltpu.make_async_copy(img_hbm.at[cidx_v.at[pl.ds(j * C, C)]],
                                  sig_v.at[pl.ds(j * C, C)], sem).wait()
            return cc
        lax.fori_loop(0, nch, _drain, 0)

        nvec = (n_valid + 15) >> 4

        def _outp(g, cc):
            sv = sig_v[pl.ds(g * 16, 16)]
            pos = cpos_v[pl.ds(g * 16, 16)]
            slot = ((pos >> 7) << 9) + 384 + (pos & 127)
            ok = (g * 16 + iota) < n_valid
            plsc.store_scatter(out_v, [slot], sv, mask=ok)
            return cc
        lax.fori_loop(0, nvec, _outp, 0)

        pltpu.sync_copy(out_v, out_hbm.at[pl.ds(4 * sbase, 4 * S)])
        return c
    lax.fori_loop(0, NSUB, _sub, 0)


def kernel(xyz, img):
    xyz_planar = jnp.transpose(xyz, (2, 0, 1)).reshape(3 * N)
    img_tiled = (img.reshape(512, 64, 8, 2, 128)
                 .transpose(0, 1, 3, 2, 4).reshape(64 * N))
    out = _ct_gather(xyz_planar, img_tiled)
    return out.reshape(N // 128, 4, 128).transpose(0, 2, 1).reshape(1, N, 4)


# speedup vs baseline: 17.3006x; 1.0003x over previous
import functools

import jax
import jax.numpy as jnp
from jax import lax
from jax.experimental import pallas as pl
from jax.experimental.pallas import tpu as pltpu
from jax.experimental.pallas import tpu_sc as plsc

N = 1048576
X_LIM, Y_LIM, Z_LIM = 511, 511, 255

NC, NS = 2, 16
NW = NC * NS
PW = N // NW
S = 8192
NSUB = PW // S
VPC = S // 16
C = 1024

_mesh = plsc.VectorSubcoreMesh(core_axis_name="c", subcore_axis_name="s")


@functools.partial(
    pl.kernel,
    mesh=_mesh,
    compiler_params=pltpu.CompilerParams(needs_layout_passes=False),
    out_type=jax.ShapeDtypeStruct((4 * N,), jnp.float32),
    scratch_types=[
        pltpu.VMEM((S,), jnp.float32),
        pltpu.VMEM((S,), jnp.float32),
        pltpu.VMEM((S,), jnp.float32),
        pltpu.VMEM((S + 16,), jnp.int32),
        pltpu.VMEM((S + 16,), jnp.int32),
        pltpu.VMEM((S,), jnp.float32),
        pltpu.VMEM((4 * S,), jnp.float32),
        pltpu.SemaphoreType.DMA,
    ],
)
def _ct_gather(xyz_hbm, img_hbm, out_hbm, x_v, y_v, z_v, cidx_v, cpos_v,
               sig_v, out_v, sem):
    wid = lax.axis_index("s") * NC + lax.axis_index("c")
    iota = lax.iota(jnp.int32, 16)
    ones16 = jnp.full((16,), 1.0, jnp.float32)
    zeros16 = jnp.full((16,), 0.0, jnp.float32)
    zeros16i = jnp.full((16,), 0, jnp.int32)

    def _fill(g, c):
        out_v[pl.ds(g * 16, 16)] = ones16
        return c
    lax.fori_loop(0, (4 * S) // 16, _fill, 0)

    def _fill0(g, c):
        cidx_v[pl.ds(g * 16, 16)] = zeros16i
        return c
    lax.fori_loop(0, (S + 16) // 16, _fill0, 0)

    def _sub(sub, c):
        sbase = wid * PW + sub * S
        pltpu.sync_copy(xyz_hbm.at[pl.ds(sbase, S)], x_v)
        pltpu.sync_copy(xyz_hbm.at[pl.ds(N + sbase, S)], y_v)
        pltpu.sync_copy(xyz_hbm.at[pl.ds(2 * N + sbase, S)], z_v)

        def _comp(g, off):
            x = x_v[pl.ds(g * 16, 16)]
            y = y_v[pl.ds(g * 16, 16)]
            z = z_v[pl.ds(g * 16, 16)]
            ix = (((x + 1.0) * 0.5) * 511.0).astype(jnp.int32)
            iy = (((y + 1.0) * 0.5) * 511.0).astype(jnp.int32)
            iz = (((z + 1.0) * 0.5) * 255.0).astype(jnp.int32)
            bad = ((ix < 0) | (iy < 0) | (iz < 0)
                   | (ix > X_LIM) | (iy > Y_LIM) | (iz > Z_LIM))
            good = jnp.logical_not(bad)
            phys = ((ix << 17) + ((iy >> 3) << 11) + ((iz >> 7) << 10)
                    + ((iy & 7) << 7) + (iz & 127))
            plsc.store_compressed(cidx_v.at[pl.ds(off, 16)], phys,
                                  mask=good)
            plsc.store_compressed(cpos_v.at[pl.ds(off, 16)], g * 16 + iota,
                                  mask=good)
            return off + jnp.max(plsc.all_reduce_population_count(good))
        n_valid = lax.fori_loop(0, VPC, _comp, jnp.int32(0))

        def _zero(g, cc):
            b = g * 16
            out_v[pl.ds((b >> 7) * 512 + 384 + (b & 127), 16)] = zeros16
            return cc
        lax.fori_loop(0, VPC, _zero, 0)

        nch = (n_valid + (C - 1)) // C

        def _fire(j, cc):
            pltpu.async_copy(img_hbm.at[cidx_v.at[pl.ds(j * C, C)]],
                             sig_v.at[pl.ds(j * C, C)], sem)
            return cc
        lax.fori_loop(0, nch, _fire, 0)

        def _drain(j, cc):
            pltpu.make_async_copy(img_hbm.at[cidx_v.at[pl.ds(j * C, C)]],
                                  sig_v.at[pl.ds(j * C, C)], sem).wait()
            return cc
        lax.fori_loop(0, nch, _drain, 0)

        nvec = (n_valid + 15) >> 4

        def _outp(g, cc):
            sv = sig_v[pl.ds(g * 16, 16)]
            pos = cpos_v[pl.ds(g * 16, 16)]
            slot = ((pos >> 7) << 9) + 384 + (pos & 127)
            ok = (g * 16 + iota) < n_valid
            plsc.store_scatter(out_v, [slot], sv, mask=ok)
            return cc
        lax.fori_loop(0, nvec, _outp, 0)

        pltpu.sync_copy(out_v, out_hbm.at[pl.ds(4 * sbase, 4 * S)])
        return c
    lax.fori_loop(0, NSUB, _sub, 0)


def kernel(xyz, img):
    xyz_planar = jnp.transpose(xyz, (2, 0, 1)).reshape(3 * N)
    img_tiled = (img.reshape(512, 64, 8, 2, 128)
                 .transpose(0, 1, 3, 2, 4).reshape(64 * N))
    out = _ct_gather(xyz_planar, img_tiled)
    return out.reshape(N // 128, 4, 128).transpose(0, 2, 1).reshape(1, N, 4)


# speedup vs baseline: 17.5116x; 1.0122x over previous
import functools

import jax
import jax.numpy as jnp
from jax import lax
from jax.experimental import pallas as pl
from jax.experimental.pallas import tpu as pltpu
from jax.experimental.pallas import tpu_sc as plsc

N = 1048576
X_LIM, Y_LIM, Z_LIM = 511, 511, 255

NC, NS = 2, 16
NW = NC * NS
PW = N // NW
S = 8192
NSUB = PW // S
VPC = S // 16
C = 512

_mesh = plsc.VectorSubcoreMesh(core_axis_name="c", subcore_axis_name="s")


@functools.partial(
    pl.kernel,
    mesh=_mesh,
    compiler_params=pltpu.CompilerParams(needs_layout_passes=False),
    out_type=jax.ShapeDtypeStruct((4 * N,), jnp.float32),
    scratch_types=[
        pltpu.VMEM((S,), jnp.float32),
        pltpu.VMEM((S,), jnp.float32),
        pltpu.VMEM((S,), jnp.float32),
        pltpu.VMEM((S + 16,), jnp.int32),
        pltpu.VMEM((S + 16,), jnp.int32),
        pltpu.VMEM((S + 16,), jnp.int32),
        pltpu.VMEM((S + 16,), jnp.int32),
        pltpu.VMEM((S,), jnp.float32),
        pltpu.VMEM((S,), jnp.float32),
        pltpu.VMEM((4 * S,), jnp.float32),
        pltpu.SemaphoreType.DMA,
        pltpu.SemaphoreType.DMA,
    ],
)
def _ct_gather(xyz_hbm, img_hbm, out_hbm, x_v, y_v, z_v, cidx_a, cidx_b,
               cpos_a, cpos_b, sig_a, sig_b, out_v, sem_a, sem_b):
    wid = lax.axis_index("s") * NC + lax.axis_index("c")
    iota = lax.iota(jnp.int32, 16)
    ones16 = jnp.full((16,), 1.0, jnp.float32)
    zeros16 = jnp.full((16,), 0.0, jnp.float32)
    zeros16i = jnp.full((16,), 0, jnp.int32)
    base = wid * PW

    def _fill(g, c):
        out_v[pl.ds(g * 16, 16)] = ones16
        return c
    lax.fori_loop(0, (4 * S) // 16, _fill, 0)

    def _fill0(g, c):
        cidx_a[pl.ds(g * 16, 16)] = zeros16i
        cidx_b[pl.ds(g * 16, 16)] = zeros16i
        return c
    lax.fori_loop(0, (S + 16) // 16, _fill0, 0)

    def _pass1(sub, cidx_v, cpos_v):
        sbase = base + sub * S
        pltpu.sync_copy(xyz_hbm.at[pl.ds(sbase, S)], x_v)
        pltpu.sync_copy(xyz_hbm.at[pl.ds(N + sbase, S)], y_v)
        pltpu.sync_copy(xyz_hbm.at[pl.ds(2 * N + sbase, S)], z_v)

        def _comp(g, off):
            x = x_v[pl.ds(g * 16, 16)]
            y = y_v[pl.ds(g * 16, 16)]
            z = z_v[pl.ds(g * 16, 16)]
            ix = ((x + 1.0) * 255.5).astype(jnp.int32)
            iy = ((y + 1.0) * 255.5).astype(jnp.int32)
            iz = ((z + 1.0) * 127.5).astype(jnp.int32)
            good = ((ix.astype(jnp.uint32) <= X_LIM)
                    & (iy.astype(jnp.uint32) <= Y_LIM)
                    & (iz.astype(jnp.uint32) <= Z_LIM))
            phys = ((ix << 17) + ((iy >> 3) << 11) + ((iz >> 7) << 10)
                    + ((iy & 7) << 7) + (iz & 127))
            plsc.store_compressed(cidx_v.at[pl.ds(off, 16)], phys,
                                  mask=good)
            plsc.store_compressed(cpos_v.at[pl.ds(off, 16)], g * 16 + iota,
                                  mask=good)
            return off + jnp.max(plsc.all_reduce_population_count(good))
        return lax.fori_loop(0, VPC, _comp, jnp.int32(0))

    def _fire(cidx_v, sig_v, sem, n_valid):
        nch = (n_valid + (C - 1)) // C

        def _f(j, cc):
            pltpu.async_copy(img_hbm.at[cidx_v.at[pl.ds(j * C, C)]],
                             sig_v.at[pl.ds(j * C, C)], sem)
            return cc
        lax.fori_loop(0, nch, _f, 0)

    def _drain(cidx_v, sig_v, sem, n_valid):
        nch = (n_valid + (C - 1)) // C

        def _d(j, cc):
            pltpu.make_async_copy(img_hbm.at[cidx_v.at[pl.ds(j * C, C)]],
                                  sig_v.at[pl.ds(j * C, C)], sem).wait()
            return cc
        lax.fori_loop(0, nch, _d, 0)

    def _pass2(sub, cpos_v, sig_v, n_valid):
        def _zero(g, cc):
            b = g * 16
            out_v[pl.ds((b >> 7) * 512 + 384 + (b & 127), 16)] = zeros16
            return cc
        lax.fori_loop(0, VPC, _zero, 0)

        nvec = (n_valid + 15) >> 4

        def _outp(g, cc):
            sv = sig_v[pl.ds(g * 16, 16)]
            pos = cpos_v[pl.ds(g * 16, 16)]
            slot = ((pos >> 7) << 9) + 384 + (pos & 127)
            ok = (g * 16 + iota) < n_valid
            plsc.store_scatter(out_v, [slot], sv, mask=ok)
            return cc
        lax.fori_loop(0, nvec, _outp, 0)

        pltpu.sync_copy(out_v, out_hbm.at[pl.ds(4 * (base + sub * S), 4 * S)])

    bufs = [(cidx_a, cpos_a, sig_a, sem_a), (cidx_b, cpos_b, sig_b, sem_b)]
    nv = [None] * NSUB
    nv[0] = _pass1(0, bufs[0][0], bufs[0][1])
    _fire(bufs[0][0], bufs[0][2], bufs[0][3], nv[0])
    for i in range(NSUB):
        if i + 1 < NSUB:
            ci, cp, sg, sm = bufs[(i + 1) % 2]
            nv[i + 1] = _pass1(i + 1, ci, cp)
            _fire(ci, sg, sm, nv[i + 1])
        ci, cp, sg, sm = bufs[i % 2]
        _drain(ci, sg, sm, nv[i])
        _pass2(i, cp, sg, nv[i])


def kernel(xyz, img):
    xyz_planar = jnp.transpose(xyz, (2, 0, 1)).reshape(3 * N)
    img_tiled = (img.reshape(512, 64, 8, 2, 128)
                 .transpose(0, 1, 3, 2, 4).reshape(64 * N))
    out = _ct_gather(xyz_planar, img_tiled)
    return out.reshape(N // 128, 4, 128).transpose(0, 2, 1).reshape(1, N, 4)
